# baseline (speedup 1.0000x reference)
import functools

import jax
import jax.numpy as jnp
from jax import lax
from jax.experimental import pallas as pl
from jax.experimental.pallas import tpu as pltpu
from jax.experimental.pallas import tpu_sc as plsc

_NC, _NS, _L = 2, 16, 16
_NW = _NC * _NS
_ROWS, _D = 8192, 4096
_RPW = _ROWS // _NW
_R = 8
_NCHUNK = _RPW // _R
_NF = _D // _L


def _sc_body(x_hbm, gidx_hbm, mask_hbm, out_hbm,
             idx_v, mask_v, ibuf, gbuf, sem_i, sem_g):
    wid = lax.axis_index("s") * _NC + lax.axis_index("c")
    wbase = wid * _RPW
    b = wid // (_NW // 4)

    pltpu.sync_copy(gidx_hbm.at[pl.ds(wbase, _RPW)], idx_v)
    pltpu.sync_copy(mask_hbm.at[b], mask_v)

    def chunk(c, carry):
        base = wbase + c * _R
        cp_i = pltpu.async_copy(x_hbm.at[pl.ds(base, _R)], ibuf, sem_i)
        cp_g = pltpu.async_copy(x_hbm.at[idx_v.at[pl.ds(c * _R, _R)]],
                                gbuf, sem_g)
        cp_i.wait()
        cp_g.wait()

        def feat(f, fcarry):
            m = mask_v[pl.ds(f * _L, _L)]
            pred = m < 0.5
            col = lax.iota(jnp.int32, _L) + f * _L
            for r in range(_R):
                g = gbuf[r, pl.ds(f * _L, _L)]
                row = jnp.full((_L,), r, dtype=jnp.int32)
                plsc.store_scatter(ibuf, [row, col], g, mask=pred)
            return fcarry

        lax.fori_loop(0, _NF, feat, 0)
        pltpu.sync_copy(ibuf, out_hbm.at[pl.ds(base, _R)])
        return carry

    lax.fori_loop(0, _NCHUNK, chunk, 0)


@functools.cache
def _build():
    mesh = plsc.VectorSubcoreMesh(core_axis_name="c", subcore_axis_name="s")
    return pl.kernel(
        _sc_body,
        out_type=jax.ShapeDtypeStruct((_ROWS, _D), jnp.float32),
        mesh=mesh,
        scratch_types=[
            pltpu.VMEM((_RPW,), jnp.int32),
            pltpu.VMEM((_D,), jnp.float32),
            pltpu.VMEM((_R, _D), jnp.float32),
            pltpu.VMEM((_R, _D), jnp.float32),
            pltpu.SemaphoreType.DMA,
            pltpu.SemaphoreType.DMA,
        ],
        compiler_params=pltpu.CompilerParams(
            use_tc_tiling_on_sc=False, needs_layout_passes=False),
    )


def kernel(x):
    bsz, seqlen, d = x.shape
    base = jax.random.key(0)
    kperm = jax.random.fold_in(base, 1)
    kmask = jax.random.fold_in(base, 2)
    permutation = jax.random.permutation(kperm, seqlen)
    area_mask = jax.random.bernoulli(kmask, 0.5, (bsz, d)).astype(x.dtype)
    gidx = (jnp.arange(bsz, dtype=jnp.int32)[:, None] * seqlen
            + permutation.astype(jnp.int32)[None, :]).reshape(-1)
    x2 = x.reshape(bsz * seqlen, d)
    out2 = _build()(x2, gidx, area_mask)
    return out2.reshape(bsz, seqlen, d)


# speedup vs baseline: 1.4897x; 1.4897x over previous
import functools

import jax
import jax.numpy as jnp
from jax import lax
from jax.experimental import pallas as pl
from jax.experimental.pallas import tpu as pltpu
from jax.experimental.pallas import tpu_sc as plsc

_NC, _NS, _L = 2, 16, 16
_NW = _NC * _NS
_ROWS, _D = 8192, 4096
_RPW = _ROWS // _NW
_R = 8
_NCHUNK = _RPW // _R
_NF = _D // _L


def _sc_body(x_hbm, gidx_hbm, mask_hbm, out_hbm,
             idx_v, mask_v, ibuf, gbuf, sem_i, sem_g):
    wid = lax.axis_index("s") * _NC + lax.axis_index("c")
    wbase = wid * _RPW
    b = wid // (_NW // 4)

    pltpu.sync_copy(gidx_hbm.at[pl.ds(wbase, _RPW)], idx_v)
    pltpu.sync_copy(mask_hbm.at[b], mask_v)

    def chunk(c, carry):
        base = wbase + c * _R
        cp_i = pltpu.async_copy(x_hbm.at[pl.ds(base, _R)], ibuf, sem_i)
        cp_g = pltpu.async_copy(x_hbm.at[idx_v.at[pl.ds(c * _R, _R)]],
                                gbuf, sem_g)
        cp_i.wait()
        cp_g.wait()

        @plsc.parallel_loop(0, _NF, unroll=2)
        def feat(f):
            m = mask_v[pl.ds(f * _L, _L)]
            pred = m < 0.5
            col = lax.iota(jnp.int32, _L) + f * _L
            for r in range(_R):
                g = gbuf[r, pl.ds(f * _L, _L)]
                row = jnp.full((_L,), r, dtype=jnp.int32)
                plsc.store_scatter(ibuf, [row, col], g, mask=pred)
        pltpu.sync_copy(ibuf, out_hbm.at[pl.ds(base, _R)])
        return carry

    lax.fori_loop(0, _NCHUNK, chunk, 0)


@functools.cache
def _build():
    mesh = plsc.VectorSubcoreMesh(core_axis_name="c", subcore_axis_name="s")
    return pl.kernel(
        _sc_body,
        out_type=jax.ShapeDtypeStruct((_ROWS, _D), jnp.float32),
        mesh=mesh,
        scratch_types=[
            pltpu.VMEM((_RPW,), jnp.int32),
            pltpu.VMEM((_D,), jnp.float32),
            pltpu.VMEM((_R, _D), jnp.float32),
            pltpu.VMEM((_R, _D), jnp.float32),
            pltpu.SemaphoreType.DMA,
            pltpu.SemaphoreType.DMA,
        ],
        compiler_params=pltpu.CompilerParams(
            use_tc_tiling_on_sc=False, needs_layout_passes=False),
    )


def kernel(x):
    bsz, seqlen, d = x.shape
    base = jax.random.key(0)
    kperm = jax.random.fold_in(base, 1)
    kmask = jax.random.fold_in(base, 2)
    permutation = jax.random.permutation(kperm, seqlen)
    area_mask = jax.random.bernoulli(kmask, 0.5, (bsz, d)).astype(x.dtype)
    gidx = (jnp.arange(bsz, dtype=jnp.int32)[:, None] * seqlen
            + permutation.astype(jnp.int32)[None, :]).reshape(-1)
    x2 = x.reshape(bsz * seqlen, d)
    out2 = _build()(x2, gidx, area_mask)
    return out2.reshape(bsz, seqlen, d)


# speedup vs baseline: 1.5689x; 1.0531x over previous
import functools

import jax
import jax.numpy as jnp
from jax import lax
from jax.experimental import pallas as pl
from jax.experimental.pallas import tpu as pltpu
from jax.experimental.pallas import tpu_sc as plsc

_NC, _NS, _L = 2, 16, 16
_NW = _NC * _NS
_ROWS, _D = 8192, 4096
_RPW = _ROWS // _NW
_R = 8
_HR = _R // 2
_NCHUNK = _RPW // _R
_NHALF = _RPW // _HR
_NF = _D // _L


def _sc_body(x_hbm, gidx_hbm, mask_hbm, out_hbm,
             idx_v, mask_v, ibuf_a, ibuf_b, gbuf_a, gbuf_b,
             sem_ia, sem_ib, sem_ga, sem_gb, sem_oa, sem_ob):
    ibufs, sem_i = (ibuf_a, ibuf_b), (sem_ia, sem_ib)
    gbufs, sem_g = (gbuf_a, gbuf_b), (sem_ga, sem_gb)
    sem_o = (sem_oa, sem_ob)

    wid = lax.axis_index("s") * _NC + lax.axis_index("c")
    wbase = wid * _RPW
    batch = wid // (_NW // 4)

    pltpu.sync_copy(gidx_hbm.at[pl.ds(wid * _NHALF, _NHALF)], idx_v)
    pltpu.sync_copy(mask_hbm.at[batch], mask_v)

    def patch(obuf, gb, h):
        @plsc.parallel_loop(0, _NF, unroll=2)
        def feat(f):
            m = mask_v[pl.ds(f * _L, _L)]
            pred = m < 0.5
            col = lax.iota(jnp.int32, _L) + f * _L
            for r in range(_HR):
                g = gb[r, pl.ds(f * _L, _L)]
                row = jnp.full((_L,), _HR * h + r, dtype=jnp.int32)
                plsc.store_scatter(obuf, [row, col], g, mask=pred)

    def issue_in(c, bs):
        base = wbase + c * _R
        pltpu.async_copy(x_hbm.at[pl.ds(base, _R)], ibufs[bs], sem_i[bs])
        pltpu.async_copy(x_hbm.at[idx_v.at[2 * c]], gbufs[0], sem_g[0])
        pltpu.async_copy(x_hbm.at[idx_v.at[2 * c + 1]], gbufs[1], sem_g[1])

    issue_in(0, 0)

    def pair(i, carry):
        for bs in (0, 1):
            c = 2 * i + bs
            base = wbase + c * _R
            nb = 1 - bs
            pltpu.make_async_copy(
                x_hbm.at[pl.ds(base, _R)], ibufs[bs], sem_i[bs]).wait()
            pltpu.make_async_copy(
                x_hbm.at[idx_v.at[2 * c]], gbufs[0], sem_g[0]).wait()
            patch(ibufs[bs], gbufs[0], 0)
            pltpu.make_async_copy(
                x_hbm.at[idx_v.at[2 * c + 1]], gbufs[1], sem_g[1]).wait()
            patch(ibufs[bs], gbufs[1], 1)
            pltpu.async_copy(ibufs[bs], out_hbm.at[pl.ds(base, _R)],
                             sem_o[bs])

            @pl.when(c + 1 < _NCHUNK)
            def _issue_next():
                @pl.when(c >= 1)
                def _drain_prev_out():
                    pltpu.make_async_copy(
                        ibufs[nb],
                        out_hbm.at[pl.ds(wbase + (c - 1) * _R, _R)],
                        sem_o[nb]).wait()
                issue_in(c + 1, nb)
        return carry

    lax.fori_loop(0, _NCHUNK // 2, pair, 0)

    pltpu.make_async_copy(
        ibufs[0], out_hbm.at[pl.ds(wbase + (_NCHUNK - 2) * _R, _R)],
        sem_o[0]).wait()
    pltpu.make_async_copy(
        ibufs[1], out_hbm.at[pl.ds(wbase + (_NCHUNK - 1) * _R, _R)],
        sem_o[1]).wait()


@functools.cache
def _build():
    mesh = plsc.VectorSubcoreMesh(core_axis_name="c", subcore_axis_name="s")
    return pl.kernel(
        _sc_body,
        out_type=jax.ShapeDtypeStruct((_ROWS, _D), jnp.float32),
        mesh=mesh,
        scratch_types=[
            pltpu.VMEM((_NHALF, _HR), jnp.int32),
            pltpu.VMEM((_D,), jnp.float32),
            pltpu.VMEM((_R, _D), jnp.float32),
            pltpu.VMEM((_R, _D), jnp.float32),
            pltpu.VMEM((_HR, _D), jnp.float32),
            pltpu.VMEM((_HR, _D), jnp.float32),
            pltpu.SemaphoreType.DMA,
            pltpu.SemaphoreType.DMA,
            pltpu.SemaphoreType.DMA,
            pltpu.SemaphoreType.DMA,
            pltpu.SemaphoreType.DMA,
            pltpu.SemaphoreType.DMA,
        ],
        compiler_params=pltpu.CompilerParams(
            use_tc_tiling_on_sc=False, needs_layout_passes=False),
    )


def kernel(x):
    bsz, seqlen, d = x.shape
    base = jax.random.key(0)
    kperm = jax.random.fold_in(base, 1)
    kmask = jax.random.fold_in(base, 2)
    permutation = jax.random.permutation(kperm, seqlen)
    area_mask = jax.random.bernoulli(kmask, 0.5, (bsz, d)).astype(x.dtype)
    gidx = (jnp.arange(bsz, dtype=jnp.int32)[:, None] * seqlen
            + permutation.astype(jnp.int32)[None, :]).reshape(_NW * _NHALF, _HR)
    x2 = x.reshape(bsz * seqlen, d)
    out2 = _build()(x2, gidx, area_mask)
    return out2.reshape(bsz, seqlen, d)


# speedup vs baseline: 2.9076x; 1.8533x over previous
import functools

import jax
import jax.numpy as jnp
from jax import lax
from jax.experimental import pallas as pl
from jax.experimental.pallas import tpu as pltpu
from jax.experimental.pallas import tpu_sc as plsc

_NC, _NS, _L = 2, 16, 16
_NW = _NC * _NS
_ROWS, _D = 8192, 4096
_RPW = _ROWS // _NW
_R = 8
_HR = _R // 2
_NCHUNK = _RPW // _R
_NHALF = _RPW // _HR
_NF = _D // _L


def _sc_body(x_hbm, gidx_hbm, mask_hbm, out_hbm,
             idx_v, mask_v, ibuf_a, ibuf_b, gbuf_a, gbuf_b,
             sem_ia, sem_ib, sem_ga, sem_gb, sem_oa, sem_ob):
    ibufs, sem_i = (ibuf_a, ibuf_b), (sem_ia, sem_ib)
    gbufs, sem_g = (gbuf_a, gbuf_b), (sem_ga, sem_gb)
    sem_o = (sem_oa, sem_ob)

    wid = lax.axis_index("s") * _NC + lax.axis_index("c")
    wbase = wid * _RPW
    batch = wid // (_NW // 4)

    pltpu.sync_copy(gidx_hbm.at[pl.ds(wid * _NHALF, _NHALF)], idx_v)
    pltpu.sync_copy(mask_hbm.at[batch], mask_v)

    def patch(obuf, gb, h):
        @plsc.parallel_loop(0, _NF, unroll=2)
        def feat(f):
            m = mask_v[pl.ds(f * _L, _L)]
            pred = m < 0.5
            col = lax.iota(jnp.int32, _L) + f * _L
            for r in range(_HR):
                g = gb[r, pl.ds(f * _L, _L)]
                row = jnp.full((_L,), _HR * h + r, dtype=jnp.int32)
                plsc.store_scatter(obuf, [row, col], g, mask=pred)

    def issue_in(c, bs):
        base = wbase + c * _R
        pltpu.async_copy(x_hbm.at[pl.ds(base, _R)], ibufs[bs], sem_i[bs])
        pltpu.async_copy(x_hbm.at[idx_v.at[2 * c]], gbufs[0], sem_g[0])
        pltpu.async_copy(x_hbm.at[idx_v.at[2 * c + 1]], gbufs[1], sem_g[1])

    issue_in(0, 0)

    def pair(i, carry):
        for bs in (0, 1):
            c = 2 * i + bs
            base = wbase + c * _R
            nb = 1 - bs
            pltpu.make_async_copy(
                x_hbm.at[pl.ds(base, _R)], ibufs[bs], sem_i[bs]).wait()
            pltpu.make_async_copy(
                x_hbm.at[idx_v.at[2 * c]], gbufs[0], sem_g[0]).wait()
            patch(ibufs[bs], gbufs[0], 0)
            pltpu.make_async_copy(
                x_hbm.at[idx_v.at[2 * c + 1]], gbufs[1], sem_g[1]).wait()
            patch(ibufs[bs], gbufs[1], 1)
            pltpu.async_copy(ibufs[bs], out_hbm.at[pl.ds(base, _R)],
                             sem_o[bs])

            @pl.when(c + 1 < _NCHUNK)
            def _issue_next():
                @pl.when(c >= 1)
                def _drain_prev_out():
                    pltpu.make_async_copy(
                        ibufs[nb],
                        out_hbm.at[pl.ds(wbase + (c - 1) * _R, _R)],
                        sem_o[nb]).wait()
                issue_in(c + 1, nb)
        return carry

    lax.fori_loop(0, _NCHUNK // 2, pair, 0)

    pltpu.make_async_copy(
        ibufs[0], out_hbm.at[pl.ds(wbase + (_NCHUNK - 2) * _R, _R)],
        sem_o[0]).wait()
    pltpu.make_async_copy(
        ibufs[1], out_hbm.at[pl.ds(wbase + (_NCHUNK - 1) * _R, _R)],
        sem_o[1]).wait()


@functools.cache
def _build():
    mesh = plsc.VectorSubcoreMesh(core_axis_name="c", subcore_axis_name="s")
    return pl.kernel(
        _sc_body,
        out_type=jax.ShapeDtypeStruct((_ROWS, _D), jnp.float32),
        mesh=mesh,
        scratch_types=[
            pltpu.VMEM((_NHALF, _HR), jnp.int32),
            pltpu.VMEM((_D,), jnp.float32),
            pltpu.VMEM((_R, _D), jnp.float32),
            pltpu.VMEM((_R, _D), jnp.float32),
            pltpu.VMEM((_HR, _D), jnp.float32),
            pltpu.VMEM((_HR, _D), jnp.float32),
            pltpu.SemaphoreType.DMA,
            pltpu.SemaphoreType.DMA,
            pltpu.SemaphoreType.DMA,
            pltpu.SemaphoreType.DMA,
            pltpu.SemaphoreType.DMA,
            pltpu.SemaphoreType.DMA,
        ],
        compiler_params=pltpu.CompilerParams(
            use_tc_tiling_on_sc=True, needs_layout_passes=False),
    )


def kernel(x):
    bsz, seqlen, d = x.shape
    base = jax.random.key(0)
    kperm = jax.random.fold_in(base, 1)
    kmask = jax.random.fold_in(base, 2)
    permutation = jax.random.permutation(kperm, seqlen)
    area_mask = jax.random.bernoulli(kmask, 0.5, (bsz, d)).astype(x.dtype)
    gidx = (jnp.arange(bsz, dtype=jnp.int32)[:, None] * seqlen
            + permutation.astype(jnp.int32)[None, :]).reshape(_NW * _NHALF, _HR)
    x2 = x.reshape(bsz * seqlen, d)
    out2 = _build()(x2, gidx, area_mask)
    return out2.reshape(bsz, seqlen, d)


# speedup vs baseline: 3.0177x; 1.0379x over previous
import functools

import jax
import jax.numpy as jnp
from jax import lax
from jax.experimental import pallas as pl
from jax.experimental.pallas import tpu as pltpu
from jax.experimental.pallas import tpu_sc as plsc

_NC, _NS, _L = 2, 16, 16
_NW = _NC * _NS
_ROWS, _D = 8192, 4096
_RPW = _ROWS // _NW
_R = 8
_HR = _R // 2
_NCHUNK = _RPW // _R
_NHALF = _RPW // _HR
_NF = _D // _L


def _sc_body(x_hbm, gidx_hbm, mask_hbm, out_hbm,
             idx_v, mask_v, ibuf_a, ibuf_b, gbuf_a, gbuf_b,
             sem_ia, sem_ib, sem_ga, sem_gb, sem_oa, sem_ob):
    ibufs, sem_i = (ibuf_a, ibuf_b), (sem_ia, sem_ib)
    gbufs, sem_g = (gbuf_a, gbuf_b), (sem_ga, sem_gb)
    sem_o = (sem_oa, sem_ob)

    wid = lax.axis_index("s") * _NC + lax.axis_index("c")
    wbase = wid * _RPW
    batch = wid // (_NW // 4)

    pltpu.sync_copy(gidx_hbm.at[pl.ds(wid * _NHALF, _NHALF)], idx_v)
    pltpu.sync_copy(mask_hbm.at[batch], mask_v)

    def patch(obuf, gb0, gb1):
        @plsc.parallel_loop(0, _NF, unroll=2)
        def feat(f):
            m = mask_v[pl.ds(f * _L, _L)]
            pred = m < 0.5
            col = lax.iota(jnp.int32, _L) + f * _L
            for h, gb in ((0, gb0), (1, gb1)):
                for r in range(_HR):
                    g = gb[r, pl.ds(f * _L, _L)]
                    row = jnp.full((_L,), _HR * h + r, dtype=jnp.int32)
                    plsc.store_scatter(obuf, [row, col], g, mask=pred)

    def issue_in(c, bs):
        base = wbase + c * _R
        pltpu.async_copy(x_hbm.at[pl.ds(base, _R)], ibufs[bs], sem_i[bs])
        pltpu.async_copy(x_hbm.at[idx_v.at[2 * c]], gbufs[0], sem_g[0])
        pltpu.async_copy(x_hbm.at[idx_v.at[2 * c + 1]], gbufs[1], sem_g[1])

    issue_in(0, 0)

    def pair(i, carry):
        for bs in (0, 1):
            c = 2 * i + bs
            base = wbase + c * _R
            nb = 1 - bs
            pltpu.make_async_copy(
                x_hbm.at[pl.ds(base, _R)], ibufs[bs], sem_i[bs]).wait()
            pltpu.make_async_copy(
                x_hbm.at[idx_v.at[2 * c]], gbufs[0], sem_g[0]).wait()
            pltpu.make_async_copy(
                x_hbm.at[idx_v.at[2 * c + 1]], gbufs[1], sem_g[1]).wait()
            patch(ibufs[bs], gbufs[0], gbufs[1])
            pltpu.async_copy(ibufs[bs], out_hbm.at[pl.ds(base, _R)],
                             sem_o[bs])

            @pl.when(c + 1 < _NCHUNK)
            def _issue_next():
                @pl.when(c >= 1)
                def _drain_prev_out():
                    pltpu.make_async_copy(
                        ibufs[nb],
                        out_hbm.at[pl.ds(wbase + (c - 1) * _R, _R)],
                        sem_o[nb]).wait()
                issue_in(c + 1, nb)
        return carry

    lax.fori_loop(0, _NCHUNK // 2, pair, 0)

    pltpu.make_async_copy(
        ibufs[0], out_hbm.at[pl.ds(wbase + (_NCHUNK - 2) * _R, _R)],
        sem_o[0]).wait()
    pltpu.make_async_copy(
        ibufs[1], out_hbm.at[pl.ds(wbase + (_NCHUNK - 1) * _R, _R)],
        sem_o[1]).wait()


@functools.cache
def _build():
    mesh = plsc.VectorSubcoreMesh(core_axis_name="c", subcore_axis_name="s")
    return pl.kernel(
        _sc_body,
        out_type=jax.ShapeDtypeStruct((_ROWS, _D), jnp.float32),
        mesh=mesh,
        scratch_types=[
            pltpu.VMEM((_NHALF, _HR), jnp.int32),
            pltpu.VMEM((_D,), jnp.float32),
            pltpu.VMEM((_R, _D), jnp.float32),
            pltpu.VMEM((_R, _D), jnp.float32),
            pltpu.VMEM((_HR, _D), jnp.float32),
            pltpu.VMEM((_HR, _D), jnp.float32),
            pltpu.SemaphoreType.DMA,
            pltpu.SemaphoreType.DMA,
            pltpu.SemaphoreType.DMA,
            pltpu.SemaphoreType.DMA,
            pltpu.SemaphoreType.DMA,
            pltpu.SemaphoreType.DMA,
        ],
        compiler_params=pltpu.CompilerParams(
            use_tc_tiling_on_sc=True, needs_layout_passes=False),
    )


def kernel(x):
    bsz, seqlen, d = x.shape
    base = jax.random.key(0)
    kperm = jax.random.fold_in(base, 1)
    kmask = jax.random.fold_in(base, 2)
    permutation = jax.random.permutation(kperm, seqlen)
    area_mask = jax.random.bernoulli(kmask, 0.5, (bsz, d)).astype(x.dtype)
    gidx = (jnp.arange(bsz, dtype=jnp.int32)[:, None] * seqlen
            + permutation.astype(jnp.int32)[None, :]).reshape(_NW * _NHALF, _HR)
    x2 = x.reshape(bsz * seqlen, d)
    out2 = _build()(x2, gidx, area_mask)
    return out2.reshape(bsz, seqlen, d)


# speedup vs baseline: 3.4120x; 1.1307x over previous
import functools

import jax
import jax.numpy as jnp
from jax import lax
from jax.experimental import pallas as pl
from jax.experimental.pallas import tpu as pltpu
from jax.experimental.pallas import tpu_sc as plsc

_NC, _NS, _L = 2, 16, 16
_NW = _NC * _NS
_ROWS, _D = 8192, 4096
_RPW = _ROWS // _NW
_R = 8
_HR = _R // 2
_NCHUNK = _RPW // _R
_NHALF = _RPW // _HR
_NF = _D // _L


def _sc_body(x_hbm, gidx_hbm, mask_hbm, out_hbm,
             idx_v, mask_v, ibuf_a, ibuf_b, gbuf_a, gbuf_b,
             sem_ia, sem_ib, sem_ga, sem_gb, sem_oa, sem_ob):
    ibufs, sem_i = (ibuf_a, ibuf_b), (sem_ia, sem_ib)
    gbufs, sem_g = (gbuf_a, gbuf_b), (sem_ga, sem_gb)
    sem_o = (sem_oa, sem_ob)

    wid = lax.axis_index("s") * _NC + lax.axis_index("c")
    wbase = wid * _RPW
    batch = wid // (_NW // 4)

    pltpu.sync_copy(gidx_hbm.at[pl.ds(wid * _NHALF, _NHALF)], idx_v)
    pltpu.sync_copy(mask_hbm.at[batch], mask_v)

    def patch(obuf, gb0, gb1):
        @plsc.parallel_loop(0, _NF, unroll=2)
        def feat(f):
            m = mask_v[pl.ds(f * _L, _L)]
            pred = m < 0.5
            col = lax.iota(jnp.int32, _L) + f * _L
            for h, gb in ((0, gb0), (1, gb1)):
                for r in range(_HR):
                    g = gb[r, pl.ds(f * _L, _L)]
                    row = jnp.full((_L,), _HR * h + r, dtype=jnp.int32)
                    plsc.store_scatter(obuf, [row, col], g, mask=pred)

    def issue_in(c, bs):
        base = wbase + c * _R
        pltpu.async_copy(x_hbm.at[pl.ds(base, _R)], ibufs[bs], sem_i[bs])
        pltpu.async_copy(x_hbm.at[idx_v.at[2 * c]], gbufs[0], sem_g[0])
        pltpu.async_copy(x_hbm.at[idx_v.at[2 * c + 1]], gbufs[1], sem_g[1])

    issue_in(0, 0)

    def pair(i, carry):
        for bs in (0, 1):
            c = 2 * i + bs
            base = wbase + c * _R
            nb = 1 - bs
            pltpu.make_async_copy(
                x_hbm.at[pl.ds(base, _R)], ibufs[bs], sem_i[bs]).wait()
            pltpu.make_async_copy(
                x_hbm.at[idx_v.at[2 * c]], gbufs[0], sem_g[0]).wait()
            pltpu.make_async_copy(
                x_hbm.at[idx_v.at[2 * c + 1]], gbufs[1], sem_g[1]).wait()
            patch(ibufs[bs], gbufs[0], gbufs[1])
            pltpu.async_copy(ibufs[bs], out_hbm.at[pl.ds(base, _R)],
                             sem_o[bs])

            @pl.when(c + 1 < _NCHUNK)
            def _issue_next():
                @pl.when(c >= 1)
                def _drain_prev_out():
                    pltpu.make_async_copy(
                        ibufs[nb],
                        out_hbm.at[pl.ds(wbase + (c - 1) * _R, _R)],
                        sem_o[nb]).wait()
                issue_in(c + 1, nb)
        return carry

    lax.fori_loop(0, _NCHUNK // 2, pair, 0)

    pltpu.make_async_copy(
        ibufs[0], out_hbm.at[pl.ds(wbase + (_NCHUNK - 2) * _R, _R)],
        sem_o[0]).wait()
    pltpu.make_async_copy(
        ibufs[1], out_hbm.at[pl.ds(wbase + (_NCHUNK - 1) * _R, _R)],
        sem_o[1]).wait()


@functools.cache
def _build():
    mesh = plsc.VectorSubcoreMesh(core_axis_name="c", subcore_axis_name="s")
    return pl.kernel(
        _sc_body,
        out_type=jax.ShapeDtypeStruct((_ROWS, _D), jnp.float32),
        mesh=mesh,
        scratch_types=[
            pltpu.VMEM((_NHALF, _HR), jnp.int32),
            pltpu.VMEM((_D,), jnp.float32),
            pltpu.VMEM((_R, _D), jnp.float32),
            pltpu.VMEM((_R, _D), jnp.float32),
            pltpu.VMEM((_HR, _D), jnp.float32),
            pltpu.VMEM((_HR, _D), jnp.float32),
            pltpu.SemaphoreType.DMA,
            pltpu.SemaphoreType.DMA,
            pltpu.SemaphoreType.DMA,
            pltpu.SemaphoreType.DMA,
            pltpu.SemaphoreType.DMA,
            pltpu.SemaphoreType.DMA,
        ],
        compiler_params=pltpu.CompilerParams(
            use_tc_tiling_on_sc=True, needs_layout_passes=False),
    )


@functools.cache
def _constants(bsz, seqlen, d):
    import numpy as np
    with jax.ensure_compile_time_eval():
        base = jax.random.key(0)
        kperm = jax.random.fold_in(base, 1)
        kmask = jax.random.fold_in(base, 2)
        permutation = jax.random.permutation(kperm, seqlen)
        area_mask = jax.random.bernoulli(
            kmask, 0.5, (bsz, d)).astype(jnp.float32)
        gidx = (jnp.arange(bsz, dtype=jnp.int32)[:, None] * seqlen
                + permutation.astype(jnp.int32)[None, :]
                ).reshape(_NW * _NHALF, _HR)
        return np.asarray(gidx), np.asarray(area_mask)


def kernel(x):
    bsz, seqlen, d = x.shape
    gidx, area_mask = _constants(bsz, seqlen, d)
    x2 = x.reshape(bsz * seqlen, d)
    out2 = _build()(x2, gidx, area_mask)
    return out2.reshape(bsz, seqlen, d)
